# Initial kernel scaffold; baseline (speedup 1.0000x reference)
import jax
import jax.numpy as jnp
from jax.experimental import pallas as pl


def kernel(x, keys_p, values_p, W_key, scale):
    raise NotImplementedError("write your pallas kernel here")


# speedup vs baseline: 104.7702x; 104.7702x over previous
import math

import jax
import jax.numpy as jnp
from jax import lax
from jax.experimental import pallas as pl
from jax.experimental.pallas import tpu as pltpu
from jax.experimental.pallas import tpu_sc as plsc

HID = 2048
KS = 64
TK = 4
NSLOTS = 100000
T = 2048
TBLK = 512
SBLK = 2048
NKB = (NSLOTS + SBLK - 1) // SBLK
PADS = NKB * SBLK

NW = 32
TPW = T // NW
CTOK = 4
NCH = TPW // CTOK


def _topk_body(scale_ref, x_ref, wkt_ref, keys_ref, w_out, i_out, q_s):
    kb = pl.program_id(1)

    @pl.when(kb == 0)
    def _init():
        q_s[...] = jnp.dot(x_ref[...], wkt_ref[...],
                           preferred_element_type=jnp.float32)
        w_out[...] = jnp.full((TBLK, 128), -jnp.inf, jnp.float32)
        i_out[...] = jnp.zeros((TBLK, 128), jnp.int32)

    s = lax.dot_general(q_s[...], keys_ref[...], (((1,), (1,)), ((), ())),
                        preferred_element_type=jnp.float32)
    s = s * (1.0 / math.sqrt(KS))
    col = lax.broadcasted_iota(jnp.int32, (TBLK, SBLK), 1) + kb * SBLK
    s = jnp.where(col < NSLOTS, s, -jnp.inf)

    aug_v = jnp.concatenate([s, w_out[...]], axis=1)
    aug_i = jnp.concatenate([col, i_out[...]], axis=1)
    nv, ni = [], []
    for _ in range(TK):
        m = jnp.max(aug_v, axis=1, keepdims=True)
        hit = aug_v == m
        sel = jnp.min(jnp.where(hit, aug_i, jnp.int32(2147483647)),
                      axis=1, keepdims=True)
        nv.append(m)
        ni.append(sel)
        aug_v = jnp.where(aug_i == sel, -jnp.inf, aug_v)
    pad_v = jnp.full((TBLK, 128 - TK), -jnp.inf, jnp.float32)
    pad_i = jnp.zeros((TBLK, 128 - TK), jnp.int32)
    w_out[...] = jnp.concatenate(nv + [pad_v], axis=1)
    i_out[...] = jnp.concatenate(ni + [pad_i], axis=1)

    @pl.when(kb == NKB - 1)
    def _finish():
        v = w_out[...]
        m = jnp.max(v, axis=1, keepdims=True)
        e = jnp.exp(v - m)
        w_out[...] = e / jnp.sum(e, axis=1, keepdims=True) * scale_ref[0]


def _topk_weights(scale1, x2d, wktT, keys_pad):
    return pl.pallas_call(
        _topk_body,
        grid=(T // TBLK, NKB),
        in_specs=[
            pl.BlockSpec(memory_space=pltpu.SMEM),
            pl.BlockSpec((TBLK, HID), lambda tb, kb: (tb, 0)),
            pl.BlockSpec((HID, KS), lambda tb, kb: (0, 0)),
            pl.BlockSpec((SBLK, KS), lambda tb, kb: (kb, 0)),
        ],
        out_specs=[
            pl.BlockSpec((TBLK, 128), lambda tb, kb: (tb, 0)),
            pl.BlockSpec((TBLK, 128), lambda tb, kb: (tb, 0)),
        ],
        out_shape=[
            jax.ShapeDtypeStruct((T, 128), jnp.float32),
            jax.ShapeDtypeStruct((T, 128), jnp.int32),
        ],
        scratch_shapes=[pltpu.VMEM((TBLK, KS), jnp.float32)],
        compiler_params=pltpu.CompilerParams(
            dimension_semantics=("arbitrary", "arbitrary")),
    )(scale1, x2d, wktT, keys_pad)


def _gather_body(vals_hbm, x_hbm, idx_hbm, w_hbm, out_hbm,
                 idx_v, w_v, rows_v, x_v, sem):
    cid = lax.axis_index("c")
    sid = lax.axis_index("s")
    wid = sid * 2 + cid

    def chunk(c, carry):
        tok0 = wid * TPW + c * CTOK
        base = tok0 * TK
        pltpu.sync_copy(idx_hbm.at[pl.ds(base, CTOK * TK)], idx_v)
        cp = pltpu.async_copy(vals_hbm.at[idx_v], rows_v, sem)
        pltpu.sync_copy(w_hbm.at[pl.ds(base, CTOK * TK)], w_v)
        pltpu.sync_copy(x_hbm.at[pl.ds(tok0, CTOK)], x_v)
        cp.wait()
        for t in range(CTOK):
            wb = [w_v[t * TK + j, :] for j in range(TK)]

            def dbody(dd, _, t=t, wb=wb):
                sl = pl.ds(dd * 16, 16)
                acc = x_v[t, sl]
                acc = acc + wb[0] * rows_v[t * TK + 0, sl]
                acc = acc + wb[1] * rows_v[t * TK + 1, sl]
                acc = acc + wb[2] * rows_v[t * TK + 2, sl]
                acc = acc + wb[3] * rows_v[t * TK + 3, sl]
                x_v[t, sl] = acc
                return 0

            lax.fori_loop(0, HID // 16, dbody, 0)
        pltpu.sync_copy(x_v, out_hbm.at[pl.ds(tok0, CTOK)])
        return carry

    lax.fori_loop(0, NCH, chunk, 0)


def _gather_combine(values_p, x2d, idx_flat, w_flat):
    mesh = plsc.VectorSubcoreMesh(core_axis_name="c", subcore_axis_name="s")
    f = pl.kernel(
        _gather_body,
        mesh=mesh,
        out_type=jax.ShapeDtypeStruct((T, HID), jnp.float32),
        scratch_types=[
            pltpu.VMEM((CTOK * TK,), jnp.int32),
            pltpu.VMEM((CTOK * TK, 16), jnp.float32),
            pltpu.VMEM((CTOK * TK, HID), jnp.float32),
            pltpu.VMEM((CTOK, HID), jnp.float32),
            pltpu.SemaphoreType.DMA,
        ],
    )
    return f(values_p, x2d, idx_flat, w_flat)


def kernel(x, keys_p, values_p, W_key, scale):
    x2d = x.reshape(T, HID)
    wktT = W_key.T
    keys_pad = jnp.pad(keys_p, ((0, PADS - NSLOTS), (0, 0)))
    scale1 = scale.reshape(1)
    w128, i128 = _topk_weights(scale1, x2d, wktT, keys_pad)
    w_bcast = jnp.broadcast_to(w128[:, :TK].reshape(-1)[:, None], (T * TK, 16))
    idx_flat = i128[:, :TK].reshape(-1)
    out = _gather_combine(values_p, x2d, idx_flat, w_bcast)
    return out.reshape(1, T, HID)


# speedup vs baseline: 121.3123x; 1.1579x over previous
import math

import jax
import jax.numpy as jnp
from jax import lax
from jax.experimental import pallas as pl
from jax.experimental.pallas import tpu as pltpu
from jax.experimental.pallas import tpu_sc as plsc

HID = 2048
KS = 64
TK = 4
NSLOTS = 100000
T = 2048
TBLK = 512
SBLK = 2048
NKB = (NSLOTS + SBLK - 1) // SBLK
PADS = NKB * SBLK

NW = 32
TPW = T // NW
CTOK = 4
NCH = TPW // CTOK


def _topk_body(scale_ref, x_ref, wkt_ref, keys_ref, w_out, i_out, q_s):
    kb = pl.program_id(1)

    @pl.when(kb == 0)
    def _init():
        q_s[...] = jnp.dot(x_ref[...], wkt_ref[...],
                           preferred_element_type=jnp.float32)
        w_out[...] = jnp.full((TBLK, 128), -jnp.inf, jnp.float32)
        i_out[...] = jnp.zeros((TBLK, 128), jnp.float32)

    s = lax.dot_general(q_s[...], keys_ref[...], (((1,), (1,)), ((), ())),
                        preferred_element_type=jnp.float32)
    s = s * (1.0 / math.sqrt(KS))
    col = lax.broadcasted_iota(jnp.int32, (TBLK, SBLK), 1) + kb * SBLK
    s = jnp.where(col < NSLOTS, s, -jnp.inf)

    colf = col.astype(jnp.float32)
    aug_v = jnp.concatenate([s, w_out[...]], axis=1)
    aug_i = jnp.concatenate([colf, i_out[...]], axis=1)
    nv, ni = [], []
    for _ in range(TK):
        m = jnp.max(aug_v, axis=1, keepdims=True)
        hit = aug_v == m
        sel = jnp.min(jnp.where(hit, aug_i, jnp.float32(3.0e7)),
                      axis=1, keepdims=True)
        nv.append(m)
        ni.append(sel)
        aug_v = jnp.where(aug_i == sel, -jnp.inf, aug_v)
    pad_v = jnp.full((TBLK, 128 - TK), -jnp.inf, jnp.float32)
    pad_i = jnp.zeros((TBLK, 128 - TK), jnp.float32)
    w_out[...] = jnp.concatenate(nv + [pad_v], axis=1)
    i_out[...] = jnp.concatenate(ni + [pad_i], axis=1)

    @pl.when(kb == NKB - 1)
    def _finish():
        v = w_out[...]
        m = jnp.max(v, axis=1, keepdims=True)
        e = jnp.exp(v - m)
        w_out[...] = e / jnp.sum(e, axis=1, keepdims=True) * scale_ref[0]


def _topk_weights(scale1, x2d, wktT, keys_pad):
    return pl.pallas_call(
        _topk_body,
        grid=(T // TBLK, NKB),
        in_specs=[
            pl.BlockSpec(memory_space=pltpu.SMEM),
            pl.BlockSpec((TBLK, HID), lambda tb, kb: (tb, 0)),
            pl.BlockSpec((HID, KS), lambda tb, kb: (0, 0)),
            pl.BlockSpec((SBLK, KS), lambda tb, kb: (kb, 0)),
        ],
        out_specs=[
            pl.BlockSpec((TBLK, 128), lambda tb, kb: (tb, 0)),
            pl.BlockSpec((TBLK, 128), lambda tb, kb: (tb, 0)),
        ],
        out_shape=[
            jax.ShapeDtypeStruct((T, 128), jnp.float32),
            jax.ShapeDtypeStruct((T, 128), jnp.float32),
        ],
        scratch_shapes=[pltpu.VMEM((TBLK, KS), jnp.float32)],
        compiler_params=pltpu.CompilerParams(
            dimension_semantics=("arbitrary", "arbitrary")),
    )(scale1, x2d, wktT, keys_pad)


def _gather_body(vals_hbm, x_hbm, idx_hbm, w_hbm, out_hbm,
                 idx_v, w_v, rows_v, x_v, sem):
    cid = lax.axis_index("c")
    sid = lax.axis_index("s")
    wid = sid * 2 + cid

    def chunk(c, carry):
        tok0 = wid * TPW + c * CTOK
        base = tok0 * TK
        pltpu.sync_copy(idx_hbm.at[pl.ds(base, CTOK * TK)], idx_v)
        cp = pltpu.async_copy(vals_hbm.at[idx_v], rows_v, sem)
        pltpu.sync_copy(w_hbm.at[pl.ds(base, CTOK * TK)], w_v)
        pltpu.sync_copy(x_hbm.at[pl.ds(tok0, CTOK)], x_v)
        cp.wait()
        for t in range(CTOK):
            wb = [w_v[t * TK + j, :] for j in range(TK)]

            def dbody(dd, _, t=t, wb=wb):
                sl = pl.ds(dd * 16, 16)
                acc = x_v[t, sl]
                acc = acc + wb[0] * rows_v[t * TK + 0, sl]
                acc = acc + wb[1] * rows_v[t * TK + 1, sl]
                acc = acc + wb[2] * rows_v[t * TK + 2, sl]
                acc = acc + wb[3] * rows_v[t * TK + 3, sl]
                x_v[t, sl] = acc
                return 0

            lax.fori_loop(0, HID // 16, dbody, 0)
        pltpu.sync_copy(x_v, out_hbm.at[pl.ds(tok0, CTOK)])
        return carry

    lax.fori_loop(0, NCH, chunk, 0)


def _gather_combine(values_p, x2d, idx_flat, w_flat):
    mesh = plsc.VectorSubcoreMesh(core_axis_name="c", subcore_axis_name="s")
    f = pl.kernel(
        _gather_body,
        mesh=mesh,
        out_type=jax.ShapeDtypeStruct((T, HID), jnp.float32),
        scratch_types=[
            pltpu.VMEM((CTOK * TK,), jnp.int32),
            pltpu.VMEM((CTOK * TK, 16), jnp.float32),
            pltpu.VMEM((CTOK * TK, HID), jnp.float32),
            pltpu.VMEM((CTOK, HID), jnp.float32),
            pltpu.SemaphoreType.DMA,
        ],
    )
    return f(values_p, x2d, idx_flat, w_flat)


def kernel(x, keys_p, values_p, W_key, scale):
    x2d = x.reshape(T, HID)
    wktT = W_key.T
    keys_pad = jnp.pad(keys_p, ((0, PADS - NSLOTS), (0, 0)))
    scale1 = scale.reshape(1)
    w128, i128 = _topk_weights(scale1, x2d, wktT, keys_pad)
    w_bcast = jnp.broadcast_to(w128[:, :TK].reshape(-1)[:, None], (T * TK, 16))
    idx_flat = i128[:, :TK].reshape(-1).astype(jnp.int32)
    out = _gather_combine(values_p, x2d, idx_flat, w_bcast)
    return out.reshape(1, T, HID)
